# baseline (speedup 1.0000x reference)
import functools

import jax
import jax.numpy as jnp
from jax import lax
from jax.experimental import pallas as pl
from jax.experimental.pallas import tpu as pltpu
from jax.experimental.pallas import tpu_sc as plsc

N = 10000
E = 160000
V = 256
H = 256
HEADS = 4
CONVS = 3

_BM = 2000

_NW = 32
_CH = 5008
_EP = _NW * _CH
_NT = HEADS * N


def _mm_kernel(x_ref, w_ref, o_ref):
    o_ref[...] = jax.lax.dot_general(
        x_ref[...], w_ref[...], (((1,), (0,)), ((), ())),
        preferred_element_type=jnp.float32)


def _mm4_kernel(x_ref, w_ref, o0, o1, o2, o3):
    y = jax.lax.dot_general(
        x_ref[...], w_ref[...], (((1,), (0,)), ((), ())),
        preferred_element_type=jnp.float32)
    o0[...] = y[:, 0 * H:1 * H]
    o1[...] = y[:, 1 * H:2 * H]
    o2[...] = y[:, 2 * H:3 * H]
    o3[...] = y[:, 3 * H:4 * H]


def _pallas_mm(x, w, bm=_BM):
    m, k = x.shape
    n = w.shape[1]
    return pl.pallas_call(
        _mm_kernel,
        grid=(m // bm,),
        in_specs=[pl.BlockSpec((bm, k), lambda i: (i, 0)),
                  pl.BlockSpec((k, n), lambda i: (0, 0))],
        out_specs=pl.BlockSpec((bm, n), lambda i: (i, 0)),
        out_shape=jax.ShapeDtypeStruct((m, n), jnp.float32),
    )(x, w)


def _pallas_mm4(x, w, bm=_BM):
    m, k = x.shape
    return pl.pallas_call(
        _mm4_kernel,
        grid=(m // bm,),
        in_specs=[pl.BlockSpec((bm, k), lambda i: (i, 0)),
                  pl.BlockSpec((k, HEADS * H), lambda i: (0, 0))],
        out_specs=[pl.BlockSpec((bm, H), lambda i: (i, 0))] * HEADS,
        out_shape=[jax.ShapeDtypeStruct((m, H), jnp.float32)] * HEADS,
    )(x, w)


_SC_MESH = plsc.VectorSubcoreMesh(core_axis_name="c", subcore_axis_name="s")


def _worker(c, s):
    return s * 2 + c


@functools.partial(
    pl.kernel, mesh=_SC_MESH,
    out_type=jax.ShapeDtypeStruct((HEADS * _EP,), jnp.float32),
    compiler_params=pltpu.CompilerParams(needs_layout_passes=False),
    scratch_types=[
        pltpu.VMEM((_NT,), jnp.float32),
        pltpu.VMEM((_NT,), jnp.float32),
        pltpu.VMEM((_CH,), jnp.int32),
        pltpu.VMEM((_CH,), jnp.int32),
        pltpu.VMEM((HEADS * _CH,), jnp.float32),
    ],
)
def _sc_edge_logits(av_h, dv_h, src_h, dst_h, e2_h, av_l, dv_l, src_l, dst_l, e_l):
    w = _worker(lax.axis_index("c"), lax.axis_index("s"))
    base = w * _CH
    pltpu.sync_copy(av_h, av_l)
    pltpu.sync_copy(dv_h, dv_l)
    pltpu.sync_copy(src_h.at[pl.ds(base, _CH)], src_l)
    pltpu.sync_copy(dst_h.at[pl.ds(base, _CH)], dst_l)

    def bstep(b, carry):
        o = b * 16
        s16 = src_l[pl.ds(o, 16)]
        d16 = dst_l[pl.ds(o, 16)]
        for h in range(HEADS):
            avv = plsc.load_gather(av_l, [s16 + h * N])
            dvv = plsc.load_gather(dv_l, [d16 + h * N])
            ev = avv + dvv
            ev = jnp.where(ev > 0, ev, ev * jnp.float32(0.01))
            e_l[pl.ds(h * _CH + o, 16)] = ev
        return carry

    lax.fori_loop(0, _CH // 16, bstep, 0)
    for h in range(HEADS):
        pltpu.sync_copy(e_l.at[pl.ds(h * _CH, _CH)], e2_h.at[pl.ds(h * _EP + base, _CH)])


@functools.partial(
    pl.kernel, mesh=_SC_MESH,
    out_type=jax.ShapeDtypeStruct((HEADS * _EP,), jnp.float32),
    compiler_params=pltpu.CompilerParams(needs_layout_passes=False),
    scratch_types=[
        pltpu.VMEM((_NT,), jnp.float32),
        pltpu.VMEM((_CH,), jnp.int32),
        pltpu.VMEM((HEADS * _CH,), jnp.float32),
    ],
)
def _sc_edge_lookup(tab_h, idx_h, g_hbm, tab_l, idx_l, g_l):
    w = _worker(lax.axis_index("c"), lax.axis_index("s"))
    base = w * _CH
    pltpu.sync_copy(tab_h, tab_l)
    pltpu.sync_copy(idx_h.at[pl.ds(base, _CH)], idx_l)

    def bstep(b, carry):
        o = b * 16
        d16 = idx_l[pl.ds(o, 16)]
        for h in range(HEADS):
            gv = plsc.load_gather(tab_l, [d16 + h * N])
            g_l[pl.ds(h * _CH + o, 16)] = gv
        return carry

    lax.fori_loop(0, _CH // 16, bstep, 0)
    for h in range(HEADS):
        pltpu.sync_copy(g_l.at[pl.ds(h * _CH, _CH)], g_hbm.at[pl.ds(h * _EP + base, _CH)])


@functools.partial(
    pl.kernel, mesh=_SC_MESH,
    out_type=tuple(jax.ShapeDtypeStruct((_EP, H), jnp.float32) for _ in range(HEADS)),
    compiler_params=pltpu.CompilerParams(needs_layout_passes=False),
    scratch_types=[
        pltpu.VMEM((HEADS * _CH,), jnp.float32),
        pltpu.VMEM((_CH,), jnp.int32),
        pltpu.VMEM((16, H), jnp.float32),
        pltpu.VMEM((16, H), jnp.float32),
        pltpu.SemaphoreType.DMA,
    ],
)
def _sc_edge_scale_rows(al_h, src_h, z0, z1, z2, z3, u0, u1, u2, u3,
                        al_l, src_l, gbuf, obuf, sem):
    w = _worker(lax.axis_index("c"), lax.axis_index("s"))
    base = w * _CH
    for h in range(HEADS):
        pltpu.sync_copy(al_h.at[pl.ds(h * _EP + base, _CH)],
                        al_l.at[pl.ds(h * _CH, _CH)])
    pltpu.sync_copy(src_h.at[pl.ds(base, _CH)], src_l)
    zts = (z0, z1, z2, z3)
    uts = (u0, u1, u2, u3)
    for h in range(HEADS):
        zt = zts[h]
        ut = uts[h]

        def bstep(b, carry):
            o = b * 16
            idx16 = src_l[pl.ds(o, 16)]
            pltpu.async_copy(zt.at[idx16], gbuf, sem).wait()
            for jj in range(16):
                wsp = plsc.load_gather(
                    al_l, [jnp.full((16,), h * _CH + jj, jnp.int32) + o])
                for c in range(H // 16):
                    obuf[jj, pl.ds(c * 16, 16)] = gbuf[jj, pl.ds(c * 16, 16)] * wsp
            pltpu.sync_copy(obuf, ut.at[pl.ds(base + o, 16)])
            return carry

        lax.fori_loop(0, _CH // 16, bstep, 0)


def kernel(wid, edge_index, emb, W0, A0, G0, B0, Wr, Ar, Gr, Br, OW, Ob):
    src = edge_index[0]
    dst = edge_index[1]
    srcp = jnp.pad(src, (0, _EP - E))
    dstp = jnp.pad(dst, (0, _EP - E))
    one_hot = jax.nn.one_hot(wid, V, dtype=jnp.float32)
    h = jnp.concatenate([one_hot, emb[wid]], axis=-1)

    for j in range(CONVS):
        if j == 0:
            Wcat = jnp.concatenate([W0[i].T for i in range(HEADS)], axis=1)
            A = A0
            G_, B_ = G0, B0
        else:
            Wcat = jnp.concatenate([Wr[j - 1, i].T for i in range(HEADS)], axis=1)
            A = Ar[j - 1]
            G_, B_ = Gr[j - 1], Br[j - 1]
        zs = _pallas_mm4(h, Wcat)

        av = jnp.concatenate([zs[i] @ A[i, :H] for i in range(HEADS)])
        dv = jnp.concatenate([zs[i] @ A[i, H:] for i in range(HEADS)])
        e2 = _sc_edge_logits(av, dv, srcp, dstp).reshape(HEADS, _EP)[:, :E]

        ms = [jax.ops.segment_max(e2[i], dst, num_segments=N) for i in range(HEADS)]
        ms = [jnp.where(jnp.isfinite(m), m, 0.0) for m in ms]
        mg = _sc_edge_lookup(jnp.concatenate(ms), dstp).reshape(HEADS, _EP)[:, :E]
        ex = [jnp.exp(e2[i] - mg[i]) for i in range(HEADS)]

        dens = [jax.ops.segment_sum(ex[i], dst, num_segments=N) for i in range(HEADS)]
        dens = [jnp.where(d > 0, d, 1.0) for d in dens]
        dg = _sc_edge_lookup(jnp.concatenate(dens), dstp).reshape(HEADS, _EP)[:, :E]

        alphas = [ex[i] / dg[i] for i in range(HEADS)]
        alpha_p = jnp.concatenate(
            [jnp.pad(alphas[i], (0, _EP - E)) for i in range(HEADS)])
        upds = _sc_edge_scale_rows(alpha_p, srcp, zs[0], zs[1], zs[2], zs[3])

        outs = []
        for i in range(HEADS):
            hn = jax.ops.segment_sum(upds[i][:E], dst, num_segments=N)
            r = jax.nn.relu(hn)
            mu = r.mean(axis=0)
            var = r.var(axis=0)
            outs.append((r - mu) / jnp.sqrt(var + 1e-5) * G_[i] + B_[i])

        h = jnp.concatenate(outs, axis=1) @ OW[j].T + Ob[j]

    return h.mean(axis=0, keepdims=True)


# speedup vs baseline: 1.1105x; 1.1105x over previous
import functools

import jax
import jax.numpy as jnp
from jax import lax
from jax.experimental import pallas as pl
from jax.experimental.pallas import tpu as pltpu
from jax.experimental.pallas import tpu_sc as plsc

N = 10000
E = 160000
V = 256
H = 256
HEADS = 4
CONVS = 3

_BM = 2000

_NW = 32
_CH = 5024
_EP = _NW * _CH
_NT = HEADS * N


def _mm_kernel(x_ref, w_ref, o_ref):
    o_ref[...] = jax.lax.dot_general(
        x_ref[...], w_ref[...], (((1,), (0,)), ((), ())),
        preferred_element_type=jnp.float32)


def _mm4_kernel(x_ref, w_ref, o0, o1, o2, o3):
    y = jax.lax.dot_general(
        x_ref[...], w_ref[...], (((1,), (0,)), ((), ())),
        preferred_element_type=jnp.float32)
    o0[...] = y[:, 0 * H:1 * H]
    o1[...] = y[:, 1 * H:2 * H]
    o2[...] = y[:, 2 * H:3 * H]
    o3[...] = y[:, 3 * H:4 * H]


def _pallas_mm(x, w, bm=_BM):
    m, k = x.shape
    n = w.shape[1]
    return pl.pallas_call(
        _mm_kernel,
        grid=(m // bm,),
        in_specs=[pl.BlockSpec((bm, k), lambda i: (i, 0)),
                  pl.BlockSpec((k, n), lambda i: (0, 0))],
        out_specs=pl.BlockSpec((bm, n), lambda i: (i, 0)),
        out_shape=jax.ShapeDtypeStruct((m, n), jnp.float32),
    )(x, w)


def _pallas_mm4(x, w, bm=_BM):
    m, k = x.shape
    return pl.pallas_call(
        _mm4_kernel,
        grid=(m // bm,),
        in_specs=[pl.BlockSpec((bm, k), lambda i: (i, 0)),
                  pl.BlockSpec((k, HEADS * H), lambda i: (0, 0))],
        out_specs=[pl.BlockSpec((bm, H), lambda i: (i, 0))] * HEADS,
        out_shape=[jax.ShapeDtypeStruct((m, H), jnp.float32)] * HEADS,
    )(x, w)


_SC_MESH = plsc.VectorSubcoreMesh(core_axis_name="c", subcore_axis_name="s")


def _worker(c, s):
    return s * 2 + c


@functools.partial(
    pl.kernel, mesh=_SC_MESH,
    out_type=jax.ShapeDtypeStruct((HEADS * _EP,), jnp.float32),
    compiler_params=pltpu.CompilerParams(needs_layout_passes=False),
    scratch_types=[
        pltpu.VMEM((_NT,), jnp.float32),
        pltpu.VMEM((_NT,), jnp.float32),
        pltpu.VMEM((_CH,), jnp.int32),
        pltpu.VMEM((_CH,), jnp.int32),
        pltpu.VMEM((HEADS * _CH,), jnp.float32),
    ],
)
def _sc_edge_logits(av_h, dv_h, src_h, dst_h, e2_h, av_l, dv_l, src_l, dst_l, e_l):
    w = _worker(lax.axis_index("c"), lax.axis_index("s"))
    base = w * _CH
    pltpu.sync_copy(av_h, av_l)
    pltpu.sync_copy(dv_h, dv_l)
    pltpu.sync_copy(src_h.at[pl.ds(base, _CH)], src_l)
    pltpu.sync_copy(dst_h.at[pl.ds(base, _CH)], dst_l)

    def bstep(b, carry):
        o = b * 16
        s16 = src_l[pl.ds(o, 16)]
        d16 = dst_l[pl.ds(o, 16)]
        for h in range(HEADS):
            avv = plsc.load_gather(av_l, [s16 + h * N])
            dvv = plsc.load_gather(dv_l, [d16 + h * N])
            ev = avv + dvv
            ev = jnp.where(ev > 0, ev, ev * jnp.float32(0.01))
            e_l[pl.ds(h * _CH + o, 16)] = ev
        return carry

    lax.fori_loop(0, _CH // 16, bstep, 0)
    for h in range(HEADS):
        pltpu.sync_copy(e_l.at[pl.ds(h * _CH, _CH)], e2_h.at[pl.ds(h * _EP + base, _CH)])


@functools.partial(
    pl.kernel, mesh=_SC_MESH,
    out_type=jax.ShapeDtypeStruct((HEADS * _EP,), jnp.float32),
    compiler_params=pltpu.CompilerParams(needs_layout_passes=False),
    scratch_types=[
        pltpu.VMEM((_NT,), jnp.float32),
        pltpu.VMEM((_CH,), jnp.int32),
        pltpu.VMEM((HEADS * _CH,), jnp.float32),
    ],
)
def _sc_edge_lookup(tab_h, idx_h, g_hbm, tab_l, idx_l, g_l):
    w = _worker(lax.axis_index("c"), lax.axis_index("s"))
    base = w * _CH
    pltpu.sync_copy(tab_h, tab_l)
    pltpu.sync_copy(idx_h.at[pl.ds(base, _CH)], idx_l)

    def bstep(b, carry):
        o = b * 16
        d16 = idx_l[pl.ds(o, 16)]
        for h in range(HEADS):
            gv = plsc.load_gather(tab_l, [d16 + h * N])
            g_l[pl.ds(h * _CH + o, 16)] = gv
        return carry

    lax.fori_loop(0, _CH // 16, bstep, 0)
    for h in range(HEADS):
        pltpu.sync_copy(g_l.at[pl.ds(h * _CH, _CH)], g_hbm.at[pl.ds(h * _EP + base, _CH)])


@functools.partial(
    pl.kernel, mesh=_SC_MESH,
    out_type=tuple(jax.ShapeDtypeStruct((_EP, H), jnp.float32) for _ in range(HEADS)),
    compiler_params=pltpu.CompilerParams(needs_layout_passes=False),
    scratch_types=[
        pltpu.VMEM((HEADS * _CH,), jnp.float32),
        pltpu.VMEM((_CH,), jnp.int32),
        pltpu.VMEM((16, H), jnp.float32),
        pltpu.VMEM((16, H), jnp.float32),
        pltpu.VMEM((16, H), jnp.float32),
        pltpu.SemaphoreType.DMA,
        pltpu.SemaphoreType.DMA,
    ],
)
def _sc_edge_scale_rows(al_h, src_h, z0, z1, z2, z3, u0, u1, u2, u3,
                        al_l, src_l, gbuf0, gbuf1, obuf, sem0, sem1):
    w = _worker(lax.axis_index("c"), lax.axis_index("s"))
    base = w * _CH
    for h in range(HEADS):
        pltpu.sync_copy(al_h.at[pl.ds(h * _EP + base, _CH)],
                        al_l.at[pl.ds(h * _CH, _CH)])
    pltpu.sync_copy(src_h.at[pl.ds(base, _CH)], src_l)
    zts = (z0, z1, z2, z3)
    uts = (u0, u1, u2, u3)
    nb = _CH // 16

    def _start(zt, b, gbuf, sem):
        pltpu.async_copy(zt.at[src_l[pl.ds(b * 16, 16)]], gbuf, sem)

    def _proc(zt, ut, b, gbuf, sem, h):
        o = b * 16
        pltpu.make_async_copy(zt.at[src_l[pl.ds(o, 16)]], gbuf, sem).wait()
        for jj in range(16):
            wsp = plsc.load_gather(
                al_l, [jnp.full((16,), h * _CH + jj, jnp.int32) + o])
            for c in range(H // 16):
                obuf[jj, pl.ds(c * 16, 16)] = gbuf[jj, pl.ds(c * 16, 16)] * wsp
        pltpu.sync_copy(obuf, ut.at[pl.ds(base + o, 16)])

    for h in range(HEADS):
        zt = zts[h]
        ut = uts[h]
        _start(zt, 0, gbuf0, sem0)

        def b2step(i, carry):
            b0 = i * 2
            _start(zt, b0 + 1, gbuf1, sem1)
            _proc(zt, ut, b0, gbuf0, sem0, h)

            @pl.when(i < nb // 2 - 1)
            def _():
                _start(zt, b0 + 2, gbuf0, sem0)

            _proc(zt, ut, b0 + 1, gbuf1, sem1, h)
            return carry

        lax.fori_loop(0, nb // 2, b2step, 0)


def kernel(wid, edge_index, emb, W0, A0, G0, B0, Wr, Ar, Gr, Br, OW, Ob):
    src = edge_index[0]
    dst = edge_index[1]
    srcp = jnp.pad(src, (0, _EP - E))
    dstp = jnp.pad(dst, (0, _EP - E))
    one_hot = jax.nn.one_hot(wid, V, dtype=jnp.float32)
    h = jnp.concatenate([one_hot, emb[wid]], axis=-1)

    for j in range(CONVS):
        if j == 0:
            Wcat = jnp.concatenate([W0[i].T for i in range(HEADS)], axis=1)
            A = A0
            G_, B_ = G0, B0
        else:
            Wcat = jnp.concatenate([Wr[j - 1, i].T for i in range(HEADS)], axis=1)
            A = Ar[j - 1]
            G_, B_ = Gr[j - 1], Br[j - 1]
        zs = _pallas_mm4(h, Wcat)

        av = jnp.concatenate([zs[i] @ A[i, :H] for i in range(HEADS)])
        dv = jnp.concatenate([zs[i] @ A[i, H:] for i in range(HEADS)])
        e2 = _sc_edge_logits(av, dv, srcp, dstp).reshape(HEADS, _EP)[:, :E]

        ms = [jax.ops.segment_max(e2[i], dst, num_segments=N) for i in range(HEADS)]
        ms = [jnp.where(jnp.isfinite(m), m, 0.0) for m in ms]
        mg = _sc_edge_lookup(jnp.concatenate(ms), dstp).reshape(HEADS, _EP)[:, :E]
        ex = [jnp.exp(e2[i] - mg[i]) for i in range(HEADS)]

        dens = [jax.ops.segment_sum(ex[i], dst, num_segments=N) for i in range(HEADS)]
        dens = [jnp.where(d > 0, d, 1.0) for d in dens]
        dg = _sc_edge_lookup(jnp.concatenate(dens), dstp).reshape(HEADS, _EP)[:, :E]

        alphas = [ex[i] / dg[i] for i in range(HEADS)]
        alpha_p = jnp.concatenate(
            [jnp.pad(alphas[i], (0, _EP - E)) for i in range(HEADS)])
        upds = _sc_edge_scale_rows(alpha_p, srcp, zs[0], zs[1], zs[2], zs[3])

        outs = []
        for i in range(HEADS):
            hn = jax.ops.segment_sum(upds[i][:E], dst, num_segments=N)
            r = jax.nn.relu(hn)
            mu = r.mean(axis=0)
            var = r.var(axis=0)
            outs.append((r - mu) / jnp.sqrt(var + 1e-5) * G_[i] + B_[i])

        h = jnp.concatenate(outs, axis=1) @ OW[j].T + Ob[j]

    return h.mean(axis=0, keepdims=True)


# speedup vs baseline: 1.2481x; 1.1239x over previous
import functools

import jax
import jax.numpy as jnp
from jax import lax
from jax.experimental import pallas as pl
from jax.experimental.pallas import tpu as pltpu
from jax.experimental.pallas import tpu_sc as plsc

N = 10000
E = 160000
V = 256
H = 256
HEADS = 4
CONVS = 3

_BM = 2000

_NW = 32
_CH = 5008
_EP = _NW * _CH
_NT = HEADS * N


def _mm_kernel(x_ref, w_ref, o_ref):
    o_ref[...] = jax.lax.dot_general(
        x_ref[...], w_ref[...], (((1,), (0,)), ((), ())),
        preferred_element_type=jnp.float32)


def _pallas_mm(x, w, bm=_BM):
    m, k = x.shape
    n = w.shape[1]
    return pl.pallas_call(
        _mm_kernel,
        grid=(m // bm,),
        in_specs=[pl.BlockSpec((bm, k), lambda i: (i, 0)),
                  pl.BlockSpec((k, n), lambda i: (0, 0))],
        out_specs=pl.BlockSpec((bm, n), lambda i: (i, 0)),
        out_shape=jax.ShapeDtypeStruct((m, n), jnp.float32),
    )(x, w)


_SC_MESH = plsc.VectorSubcoreMesh(core_axis_name="c", subcore_axis_name="s")


def _worker(c, s):
    return s * 2 + c


@functools.partial(
    pl.kernel, mesh=_SC_MESH,
    out_type=jax.ShapeDtypeStruct((HEADS * _EP,), jnp.float32),
    compiler_params=pltpu.CompilerParams(needs_layout_passes=False),
    scratch_types=[
        pltpu.VMEM((_NT,), jnp.float32),
        pltpu.VMEM((_NT,), jnp.float32),
        pltpu.VMEM((_CH,), jnp.int32),
        pltpu.VMEM((_CH,), jnp.int32),
        pltpu.VMEM((HEADS * _CH,), jnp.float32),
    ],
)
def _sc_edge_logits(av_h, dv_h, src_h, dst_h, e2_h, av_l, dv_l, src_l, dst_l, e_l):
    w = _worker(lax.axis_index("c"), lax.axis_index("s"))
    base = w * _CH
    pltpu.sync_copy(av_h, av_l)
    pltpu.sync_copy(dv_h, dv_l)
    pltpu.sync_copy(src_h.at[pl.ds(base, _CH)], src_l)
    pltpu.sync_copy(dst_h.at[pl.ds(base, _CH)], dst_l)

    def bstep(b, carry):
        o = b * 16
        s16 = src_l[pl.ds(o, 16)]
        d16 = dst_l[pl.ds(o, 16)]
        for h in range(HEADS):
            avv = plsc.load_gather(av_l, [s16 + h * N])
            dvv = plsc.load_gather(dv_l, [d16 + h * N])
            ev = avv + dvv
            ev = jnp.where(ev > 0, ev, ev * jnp.float32(0.01))
            e_l[pl.ds(h * _CH + o, 16)] = ev
        return carry

    lax.fori_loop(0, _CH // 16, bstep, 0)
    for h in range(HEADS):
        pltpu.sync_copy(e_l.at[pl.ds(h * _CH, _CH)], e2_h.at[pl.ds(h * _EP + base, _CH)])


@functools.partial(
    pl.kernel, mesh=_SC_MESH,
    out_type=jax.ShapeDtypeStruct((HEADS * _EP,), jnp.float32),
    compiler_params=pltpu.CompilerParams(needs_layout_passes=False),
    scratch_types=[
        pltpu.VMEM((_NT,), jnp.float32),
        pltpu.VMEM((_CH,), jnp.int32),
        pltpu.VMEM((HEADS * _CH,), jnp.float32),
    ],
)
def _sc_edge_lookup(tab_h, idx_h, g_hbm, tab_l, idx_l, g_l):
    w = _worker(lax.axis_index("c"), lax.axis_index("s"))
    base = w * _CH
    pltpu.sync_copy(tab_h, tab_l)
    pltpu.sync_copy(idx_h.at[pl.ds(base, _CH)], idx_l)

    def bstep(b, carry):
        o = b * 16
        d16 = idx_l[pl.ds(o, 16)]
        for h in range(HEADS):
            gv = plsc.load_gather(tab_l, [d16 + h * N])
            g_l[pl.ds(h * _CH + o, 16)] = gv
        return carry

    lax.fori_loop(0, _CH // 16, bstep, 0)
    for h in range(HEADS):
        pltpu.sync_copy(g_l.at[pl.ds(h * _CH, _CH)], g_hbm.at[pl.ds(h * _EP + base, _CH)])


def kernel(wid, edge_index, emb, W0, A0, G0, B0, Wr, Ar, Gr, Br, OW, Ob):
    src = edge_index[0]
    dst = edge_index[1]
    srcp = jnp.pad(src, (0, _EP - E))
    dstp = jnp.pad(dst, (0, _EP - E))
    one_hot = jax.nn.one_hot(wid, V, dtype=jnp.float32)
    h = jnp.concatenate([one_hot, emb[wid]], axis=-1)

    for j in range(CONVS):
        if j == 0:
            Wcat = jnp.concatenate([W0[i].T for i in range(HEADS)], axis=1)
            A = A0
            G_, B_ = G0, B0
        else:
            Wcat = jnp.concatenate([Wr[j - 1, i].T for i in range(HEADS)], axis=1)
            A = Ar[j - 1]
            G_, B_ = Gr[j - 1], Br[j - 1]
        z_all = _pallas_mm(h, Wcat)
        zs = [z_all[:, i * H:(i + 1) * H] for i in range(HEADS)]

        av = jnp.concatenate([zs[i] @ A[i, :H] for i in range(HEADS)])
        dv = jnp.concatenate([zs[i] @ A[i, H:] for i in range(HEADS)])
        e2 = _sc_edge_logits(av, dv, srcp, dstp).reshape(HEADS, _EP)[:, :E]

        ms = [jax.ops.segment_max(e2[i], dst, num_segments=N) for i in range(HEADS)]
        ms = [jnp.where(jnp.isfinite(m), m, 0.0) for m in ms]
        mg = _sc_edge_lookup(jnp.concatenate(ms), dstp).reshape(HEADS, _EP)[:, :E]
        ex = [jnp.exp(e2[i] - mg[i]) for i in range(HEADS)]

        dens = [jax.ops.segment_sum(ex[i], dst, num_segments=N) for i in range(HEADS)]
        dens = [jnp.where(d > 0, d, 1.0) for d in dens]
        dg = _sc_edge_lookup(jnp.concatenate(dens), dstp).reshape(HEADS, _EP)[:, :E]

        outs = []
        for i in range(HEADS):
            alpha = ex[i] / dg[i]
            hn = jax.ops.segment_sum(alpha[:, None] * zs[i][src], dst, num_segments=N)
            r = jax.nn.relu(hn)
            mu = r.mean(axis=0)
            var = r.var(axis=0)
            outs.append((r - mu) / jnp.sqrt(var + 1e-5) * G_[i] + B_[i])

        h = jnp.concatenate(outs, axis=1) @ OW[j].T + Ob[j]

    return h.mean(axis=0, keepdims=True)


# speedup vs baseline: 1.2490x; 1.0007x over previous
import functools

import jax
import jax.numpy as jnp
from jax import lax
from jax.experimental import pallas as pl
from jax.experimental.pallas import tpu as pltpu
from jax.experimental.pallas import tpu_sc as plsc

N = 10000
E = 160000
V = 256
H = 256
HEADS = 4
CONVS = 3

_BM = 2000

_NW = 32
_CH = 5008
_EP = _NW * _CH
_NT = HEADS * N


def _mm_kernel(x_ref, w_ref, o_ref):
    o_ref[...] = jax.lax.dot_general(
        x_ref[...], w_ref[...], (((1,), (0,)), ((), ())),
        preferred_element_type=jnp.float32)


def _pallas_mm(x, w, bm=_BM):
    m, k = x.shape
    n = w.shape[1]
    return pl.pallas_call(
        _mm_kernel,
        grid=(m // bm,),
        in_specs=[pl.BlockSpec((bm, k), lambda i: (i, 0)),
                  pl.BlockSpec((k, n), lambda i: (0, 0))],
        out_specs=pl.BlockSpec((bm, n), lambda i: (i, 0)),
        out_shape=jax.ShapeDtypeStruct((m, n), jnp.float32),
    )(x, w)


def _sc_mesh():
    return plsc.VectorSubcoreMesh(core_axis_name="c", subcore_axis_name="s")


def _worker(c, s):
    return s * 2 + c


@functools.cache
def _sc_edge_logits_kernel():
    return functools.partial(
        pl.kernel, mesh=_sc_mesh(),
        out_type=jax.ShapeDtypeStruct((HEADS * _EP,), jnp.float32),
        compiler_params=pltpu.CompilerParams(needs_layout_passes=False),
        scratch_types=[
            pltpu.VMEM((_NT,), jnp.float32),
            pltpu.VMEM((_NT,), jnp.float32),
            pltpu.VMEM((_CH,), jnp.int32),
            pltpu.VMEM((_CH,), jnp.int32),
            pltpu.VMEM((HEADS * _CH,), jnp.float32),
        ],
    )(_sc_edge_logits)


def _sc_edge_logits(av_h, dv_h, src_h, dst_h, e2_h, av_l, dv_l, src_l, dst_l, e_l):
    w = _worker(lax.axis_index("c"), lax.axis_index("s"))
    base = w * _CH
    pltpu.sync_copy(av_h, av_l)
    pltpu.sync_copy(dv_h, dv_l)
    pltpu.sync_copy(src_h.at[pl.ds(base, _CH)], src_l)
    pltpu.sync_copy(dst_h.at[pl.ds(base, _CH)], dst_l)

    def bstep(b, carry):
        o = b * 16
        s16 = src_l[pl.ds(o, 16)]
        d16 = dst_l[pl.ds(o, 16)]
        for h in range(HEADS):
            avv = plsc.load_gather(av_l, [s16 + h * N])
            dvv = plsc.load_gather(dv_l, [d16 + h * N])
            ev = avv + dvv
            ev = jnp.where(ev > 0, ev, ev * jnp.float32(0.01))
            e_l[pl.ds(h * _CH + o, 16)] = ev
        return carry

    lax.fori_loop(0, _CH // 16, bstep, 0)
    for h in range(HEADS):
        pltpu.sync_copy(e_l.at[pl.ds(h * _CH, _CH)], e2_h.at[pl.ds(h * _EP + base, _CH)])


@functools.cache
def _sc_edge_lookup_kernel():
    return functools.partial(
        pl.kernel, mesh=_sc_mesh(),
        out_type=jax.ShapeDtypeStruct((HEADS * _EP,), jnp.float32),
        compiler_params=pltpu.CompilerParams(needs_layout_passes=False),
        scratch_types=[
            pltpu.VMEM((_NT,), jnp.float32),
            pltpu.VMEM((_CH,), jnp.int32),
            pltpu.VMEM((HEADS * _CH,), jnp.float32),
        ],
    )(_sc_edge_lookup)


def _sc_edge_lookup(tab_h, idx_h, g_hbm, tab_l, idx_l, g_l):
    w = _worker(lax.axis_index("c"), lax.axis_index("s"))
    base = w * _CH
    pltpu.sync_copy(tab_h, tab_l)
    pltpu.sync_copy(idx_h.at[pl.ds(base, _CH)], idx_l)

    def bstep(b, carry):
        o = b * 16
        d16 = idx_l[pl.ds(o, 16)]
        for h in range(HEADS):
            gv = plsc.load_gather(tab_l, [d16 + h * N])
            g_l[pl.ds(h * _CH + o, 16)] = gv
        return carry

    lax.fori_loop(0, _CH // 16, bstep, 0)
    for h in range(HEADS):
        pltpu.sync_copy(g_l.at[pl.ds(h * _CH, _CH)], g_hbm.at[pl.ds(h * _EP + base, _CH)])


def kernel(wid, edge_index, emb, W0, A0, G0, B0, Wr, Ar, Gr, Br, OW, Ob):
    src = edge_index[0]
    dst = edge_index[1]
    srcp = jnp.pad(src, (0, _EP - E))
    dstp = jnp.pad(dst, (0, _EP - E))
    one_hot = jax.nn.one_hot(wid, V, dtype=jnp.float32)
    h = jnp.concatenate([one_hot, emb[wid]], axis=-1)

    for j in range(CONVS):
        if j == 0:
            Wcat = jnp.concatenate([W0[i].T for i in range(HEADS)], axis=1)
            A = A0
            G_, B_ = G0, B0
        else:
            Wcat = jnp.concatenate([Wr[j - 1, i].T for i in range(HEADS)], axis=1)
            A = Ar[j - 1]
            G_, B_ = Gr[j - 1], Br[j - 1]
        z_all = _pallas_mm(h, Wcat)
        zs = [z_all[:, i * H:(i + 1) * H] for i in range(HEADS)]

        av = jnp.concatenate([zs[i] @ A[i, :H] for i in range(HEADS)])
        dv = jnp.concatenate([zs[i] @ A[i, H:] for i in range(HEADS)])
        e2 = _sc_edge_logits_kernel()(av, dv, srcp, dstp).reshape(HEADS, _EP)[:, :E]

        ms = [jax.ops.segment_max(e2[i], dst, num_segments=N) for i in range(HEADS)]
        ms = [jnp.where(jnp.isfinite(m), m, 0.0) for m in ms]
        mg = _sc_edge_lookup_kernel()(jnp.concatenate(ms), dstp).reshape(HEADS, _EP)[:, :E]
        ex = [jnp.exp(e2[i] - mg[i]) for i in range(HEADS)]

        dens = [jax.ops.segment_sum(ex[i], dst, num_segments=N) for i in range(HEADS)]
        dens = [jnp.where(d > 0, d, 1.0) for d in dens]
        dg = _sc_edge_lookup_kernel()(jnp.concatenate(dens), dstp).reshape(HEADS, _EP)[:, :E]

        outs = []
        for i in range(HEADS):
            alpha = ex[i] / dg[i]
            hn = jax.ops.segment_sum(alpha[:, None] * zs[i][src], dst, num_segments=N)
            r = jax.nn.relu(hn)
            mu = r.mean(axis=0)
            var = r.var(axis=0)
            outs.append((r - mu) / jnp.sqrt(var + 1e-5) * G_[i] + B_[i])

        h = jnp.concatenate(outs, axis=1) @ OW[j].T + Ob[j]

    return h.mean(axis=0, keepdims=True)
